# baseline (speedup 1.0000x reference)
import functools

import numpy as np
import jax
import jax.numpy as jnp
from jax import lax
from jax.experimental import pallas as pl
from jax.experimental.pallas import tpu as pltpu
from jax.experimental.pallas import tpu_sc as plsc

_NBINS = 100
_CUTOFF = 0.35
_NA = 500
_NAP = 512
_W = _CUTOFF / (_NBINS - 1)
_INVW = (_NBINS - 1) / _CUTOFF
_J = 4
_NH = 128
_R2T = (_CUTOFF + _J * _W) ** 2
_NW = 32
_REG = 544
_NAOS = 3 * _NA * 2
_SOA = 2 * _NAP

_mesh = plsc.VectorSubcoreMesh(core_axis_name="c", subcore_axis_name="s")


@functools.partial(
    pl.kernel,
    out_type=jax.ShapeDtypeStruct((_NW * _NH,), jnp.float32),
    mesh=_mesh,
    compiler_params=pltpu.CompilerParams(needs_layout_passes=False),
    scratch_types=[
        pltpu.VMEM((_NAOS,), jnp.float32),
        pltpu.VMEM((3 * _SOA + 16,), jnp.float32),
        pltpu.VMEM((16 * _REG,), jnp.float32),
        pltpu.VMEM((16,), jnp.int32),
        pltpu.VMEM((16 * _NH,), jnp.float32),
        pltpu.VMEM((_NH,), jnp.float32),
    ],
)
def _sc_hist(coords_hbm, out_hbm, cvm, soa, buf, cntv, hist, outv):
    wid = lax.axis_index("s") * 2 + lax.axis_index("c")
    pltpu.sync_copy(coords_hbm, cvm)
    iota = lax.iota(jnp.int32, 16)
    iota3 = iota * 3
    zero16 = jnp.zeros((16,), jnp.float32)

    def zh(k, carry):
        hist[pl.ds(k * 16, 16)] = zero16
        return carry

    lax.fori_loop(0, 16 * _NH // 16, zh, 0)

    def tr(k, carry):
        p = k // 64
        b = (k // 32) % 2
        c = k % 32
        gi = iota3 + (b * 3 * _NA + c * 48 + p)
        gi = jnp.minimum(gi, _NAOS - 1)
        soa[pl.ds(p * _SOA + b * _NAP + c * 16, 16)] = (
            plsc.load_gather(cvm, [gi]))
        return carry

    lax.fori_loop(0, 192, tr, 0)

    def wrap_sq(d):
        w = d - (2.0 * d).astype(jnp.int32).astype(jnp.float32)
        return w * w

    lane_base = iota * _REG

    def one_batch(b, cnt0):
        base = b * _NAP
        nrows = (_NA - 1 - wid) // _NW + 1
        nrows2 = (nrows + 1) // 2

        def row_body(ri, cnt):
            ia = wid + 2 * _NW * ri
            ib = ia + _NW
            has_b = ib <= _NA - 1
            civ = jnp.full((16,), base + ia, jnp.int32)
            xa = plsc.load_gather(soa, [civ])
            ya = plsc.load_gather(soa, [civ + _SOA])
            za = plsc.load_gather(soa, [civ + 2 * _SOA])
            cjv = jnp.full((16,), base + jnp.where(has_b, ib, ia), jnp.int32)
            xb = plsc.load_gather(soa, [cjv])
            yb = plsc.load_gather(soa, [cjv + _SOA])
            zb = plsc.load_gather(soa, [cjv + 2 * _SOA])
            ihi = jnp.where(has_b, ib, ia)
            nj4 = (ihi + 63) // 64

            def jv_body(jv4, cnt1):
                off = base + jv4 * 64
                ja0 = jv4 * 64 + iota
                c = cnt1
                for s in range(4):
                    cx = soa[pl.ds(off + s * 16, 16)]
                    cy = soa[pl.ds(off + _SOA + s * 16, 16)]
                    cz = soa[pl.ds(off + 2 * _SOA + s * 16, 16)]
                    ja = ja0 + s * 16
                    da = (wrap_sq(xa - cx) + wrap_sq(ya - cy)
                          + wrap_sq(za - cz))
                    db = (wrap_sq(xb - cx) + wrap_sq(yb - cy)
                          + wrap_sq(zb - cz))
                    ma = (da < _R2T) & (da != 0.0) & (ja < ia)
                    mb = (db < _R2T) & (db != 0.0) & (ja < ib) & has_b
                    plsc.store_scatter(buf, [lane_base + c], da, mask=ma)
                    c = c + ma.astype(jnp.int32)
                    plsc.store_scatter(buf, [lane_base + c], db, mask=mb)
                    c = c + mb.astype(jnp.int32)
                return c

            return lax.fori_loop(0, nj4, jv_body, cnt)

        return lax.fori_loop(0, nrows2, row_body, cnt0)

    cnt = one_batch(0, jnp.zeros((16,), jnp.int32))
    cnt = one_batch(1, cnt)
    cntv[pl.ds(0, 16)] = cnt


    def smear(dsq, valid):
        bits = plsc.bitcast(dsq, jnp.int32)
        y = plsc.bitcast(
            jnp.int32(0x5F3759DF) - lax.shift_right_logical(bits, 1),
            jnp.float32)
        for _ in range(3):
            y = y * (1.5 - 0.5 * dsq * y * y)
        t = dsq * y * _INVW
        i0 = (t + 0.5).astype(jnp.int32)
        i0 = jnp.minimum(jnp.maximum(i0, 0), _NBINS + _J)
        f = t - i0.astype(jnp.float32)
        base_idx = iota * _NH + i0
        for jj in range(2 * _J + 1):
            a = f + float(_J - jj)
            wv = jnp.exp(-0.5 * a * a)
            plsc.addupdate_scatter(hist, [base_idx + jj], wv, mask=valid)

    def lane_loop(l, carry):
        clv = plsc.load_gather(cntv, [jnp.full((16,), l, jnp.int32)])
        nvl2 = (clv[0] + 31) // 32
        lb = l * _REG

        def pv(v, carry2):
            off = lb + v * 32
            dsq_a = buf[pl.ds(off, 16)]
            dsq_b = buf[pl.ds(off + 16, 16)]
            smear(dsq_a, (v * 32 + iota) < clv)
            smear(dsq_b, (v * 32 + 16 + iota) < clv)
            return carry2

        return lax.fori_loop(0, nvl2, pv, carry)

    lax.fori_loop(0, 16, lane_loop, jnp.int32(0))

    def red(c, carry):
        acc = hist[pl.ds(c * 16, 16)]
        for r in range(1, 16):
            acc = acc + hist[pl.ds(r * _NH + c * 16, 16)]
        outv[pl.ds(c * 16, 16)] = acc
        return carry

    lax.fori_loop(0, 8, red, 0)
    pltpu.sync_copy(outv, out_hbm.at[pl.ds(wid * _NH, _NH)])


def kernel(xyz):
    coords = xyz.reshape(-1)
    part = _sc_hist(coords).reshape(_NW, _NH)
    count = part.sum(axis=0)[_J:_J + _NBINS]
    bins = jnp.linspace(0.0, _CUTOFF, _NBINS + 1)
    vol_bins = 4.0 * np.pi / 3.0 * (bins[1:] ** 3 - bins[:-1] ** 3)
    norm = count.sum()
    count = count / norm
    V = 4.0 / 3.0 * np.pi * _CUTOFF ** 3
    rdf_out = count / (vol_bins / V)
    return (count, bins, rdf_out)


# speedup vs baseline: 1.0678x; 1.0678x over previous
import functools

import numpy as np
import jax
import jax.numpy as jnp
from jax import lax
from jax.experimental import pallas as pl
from jax.experimental.pallas import tpu as pltpu
from jax.experimental.pallas import tpu_sc as plsc

_NBINS = 100
_CUTOFF = 0.35
_NA = 500
_NAP = 512
_W = _CUTOFF / (_NBINS - 1)
_INVW = (_NBINS - 1) / _CUTOFF
_J = 4
_NH = 128
_R2T = (_CUTOFF + _J * _W) ** 2
_NW = 32
_REG = 544
_NAOS = 3 * _NA * 2
_SOA = 2 * _NAP

_mesh = plsc.VectorSubcoreMesh(core_axis_name="c", subcore_axis_name="s")


@functools.partial(
    pl.kernel,
    out_type=jax.ShapeDtypeStruct((_NW * _NH,), jnp.float32),
    mesh=_mesh,
    compiler_params=pltpu.CompilerParams(needs_layout_passes=False),
    scratch_types=[
        pltpu.VMEM((_NAOS,), jnp.float32),
        pltpu.VMEM((3 * _SOA + 16,), jnp.float32),
        pltpu.VMEM((16 * _REG,), jnp.float32),
        pltpu.VMEM((16,), jnp.int32),
        pltpu.VMEM((16 * _NH,), jnp.float32),
        pltpu.VMEM((_NH,), jnp.float32),
    ],
)
def _sc_hist(coords_hbm, out_hbm, cvm, soa, buf, cntv, hist, outv):
    wid = lax.axis_index("s") * 2 + lax.axis_index("c")
    pltpu.sync_copy(coords_hbm, cvm)
    iota = lax.iota(jnp.int32, 16)
    iota3 = iota * 3
    zero16 = jnp.zeros((16,), jnp.float32)

    def zh(k, carry):
        hist[pl.ds(k * 16, 16)] = zero16
        return carry

    lax.fori_loop(0, 16 * _NH // 16, zh, 0)

    def tr(k, carry):
        p = k // 64
        b = (k // 32) % 2
        c = k % 32
        gi = iota3 + (b * 3 * _NA + c * 48 + p)
        gi = jnp.minimum(gi, _NAOS - 1)
        soa[pl.ds(p * _SOA + b * _NAP + c * 16, 16)] = (
            plsc.load_gather(cvm, [gi]))
        return carry

    lax.fori_loop(0, 192, tr, 0)

    def wrap_sq(d):
        a = jnp.abs(d)
        b = 1.0 - a
        return jnp.minimum(a * a, b * b)

    lane_base = iota * _REG

    def one_batch(b, cnt0):
        base = b * _NAP
        nrows = (_NA - 1 - wid) // _NW + 1
        nrows2 = (nrows + 1) // 2

        def row_body(ri, cnt):
            ia = wid + 2 * _NW * ri
            ib = ia + _NW
            has_b = ib <= _NA - 1
            civ = jnp.full((16,), base + ia, jnp.int32)
            xa = plsc.load_gather(soa, [civ])
            ya = plsc.load_gather(soa, [civ + _SOA])
            za = plsc.load_gather(soa, [civ + 2 * _SOA])
            cjv = jnp.full((16,), base + jnp.where(has_b, ib, ia), jnp.int32)
            xb = plsc.load_gather(soa, [cjv])
            yb = plsc.load_gather(soa, [cjv + _SOA])
            zb = plsc.load_gather(soa, [cjv + 2 * _SOA])
            ihi = jnp.where(has_b, ib, ia)
            nj2 = (ihi + 31) // 32

            def jv_body(jv2, cnt1):
                off = base + jv2 * 32
                ja = jv2 * 32 + iota
                cx0 = soa[pl.ds(off, 16)]
                cx1 = soa[pl.ds(off + 16, 16)]
                cy0 = soa[pl.ds(off + _SOA, 16)]
                cy1 = soa[pl.ds(off + _SOA + 16, 16)]
                cz0 = soa[pl.ds(off + 2 * _SOA, 16)]
                cz1 = soa[pl.ds(off + 2 * _SOA + 16, 16)]
                d_a0 = wrap_sq(xa - cx0) + wrap_sq(ya - cy0) + wrap_sq(za - cz0)
                d_a1 = wrap_sq(xa - cx1) + wrap_sq(ya - cy1) + wrap_sq(za - cz1)
                d_b0 = wrap_sq(xb - cx0) + wrap_sq(yb - cy0) + wrap_sq(zb - cz0)
                d_b1 = wrap_sq(xb - cx1) + wrap_sq(yb - cy1) + wrap_sq(zb - cz1)
                m_a0 = (d_a0 < _R2T) & (d_a0 != 0.0) & (ja < ia)
                m_a1 = (d_a1 < _R2T) & (d_a1 != 0.0) & (ja + 16 < ia)
                m_b0 = (d_b0 < _R2T) & (d_b0 != 0.0) & (ja < ib) & has_b
                m_b1 = (d_b1 < _R2T) & (d_b1 != 0.0) & (ja + 16 < ib) & has_b
                plsc.store_scatter(buf, [lane_base + cnt1], d_a0, mask=m_a0)
                c2 = cnt1 + m_a0.astype(jnp.int32)
                plsc.store_scatter(buf, [lane_base + c2], d_a1, mask=m_a1)
                c3 = c2 + m_a1.astype(jnp.int32)
                plsc.store_scatter(buf, [lane_base + c3], d_b0, mask=m_b0)
                c4 = c3 + m_b0.astype(jnp.int32)
                plsc.store_scatter(buf, [lane_base + c4], d_b1, mask=m_b1)
                return c4 + m_b1.astype(jnp.int32)

            return lax.fori_loop(0, nj2, jv_body, cnt)

        return lax.fori_loop(0, nrows2, row_body, cnt0)

    cnt = one_batch(0, jnp.zeros((16,), jnp.int32))
    cnt = one_batch(1, cnt)
    cntv[pl.ds(0, 16)] = cnt


    def smear(dsq, valid):
        bits = plsc.bitcast(dsq, jnp.int32)
        y = plsc.bitcast(
            jnp.int32(0x5F3759DF) - lax.shift_right_logical(bits, 1),
            jnp.float32)
        for _ in range(3):
            y = y * (1.5 - 0.5 * dsq * y * y)
        t = dsq * y * _INVW
        i0 = (t + 0.5).astype(jnp.int32)
        i0 = jnp.minimum(jnp.maximum(i0, 0), _NBINS + _J)
        f = t - i0.astype(jnp.float32)
        base_idx = iota * _NH + i0
        for jj in range(2 * _J + 1):
            a = f + float(_J - jj)
            wv = jnp.exp(-0.5 * a * a)
            plsc.addupdate_scatter(hist, [base_idx + jj], wv, mask=valid)

    def lane_loop(l, carry):
        clv = plsc.load_gather(cntv, [jnp.full((16,), l, jnp.int32)])
        nvl2 = (clv[0] + 31) // 32
        lb = l * _REG

        def pv(v, carry2):
            off = lb + v * 32
            dsq_a = buf[pl.ds(off, 16)]
            dsq_b = buf[pl.ds(off + 16, 16)]
            smear(dsq_a, (v * 32 + iota) < clv)
            smear(dsq_b, (v * 32 + 16 + iota) < clv)
            return carry2

        return lax.fori_loop(0, nvl2, pv, carry)

    lax.fori_loop(0, 16, lane_loop, jnp.int32(0))

    def red(c, carry):
        acc = hist[pl.ds(c * 16, 16)]
        for r in range(1, 16):
            acc = acc + hist[pl.ds(r * _NH + c * 16, 16)]
        outv[pl.ds(c * 16, 16)] = acc
        return carry

    lax.fori_loop(0, 8, red, 0)
    pltpu.sync_copy(outv, out_hbm.at[pl.ds(wid * _NH, _NH)])


def kernel(xyz):
    coords = xyz.reshape(-1)
    part = _sc_hist(coords).reshape(_NW, _NH)
    count = part.sum(axis=0)[_J:_J + _NBINS]
    bins = jnp.linspace(0.0, _CUTOFF, _NBINS + 1)
    vol_bins = 4.0 * np.pi / 3.0 * (bins[1:] ** 3 - bins[:-1] ** 3)
    norm = count.sum()
    count = count / norm
    V = 4.0 / 3.0 * np.pi * _CUTOFF ** 3
    rdf_out = count / (vol_bins / V)
    return (count, bins, rdf_out)


# speedup vs baseline: 1.1256x; 1.0541x over previous
import functools

import numpy as np
import jax
import jax.numpy as jnp
from jax import lax
from jax.experimental import pallas as pl
from jax.experimental.pallas import tpu as pltpu
from jax.experimental.pallas import tpu_sc as plsc

_NBINS = 100
_CUTOFF = 0.35
_NA = 500
_NAP = 512
_W = _CUTOFF / (_NBINS - 1)
_INVW = (_NBINS - 1) / _CUTOFF
_J = 4
_NH = 128
_R2T = (_CUTOFF + _J * _W) ** 2
_NW = 32
_REG = 544
_NAOS = 3 * _NA * 2
_SOA = 2 * _NAP

_mesh = plsc.VectorSubcoreMesh(core_axis_name="c", subcore_axis_name="s")


@functools.partial(
    pl.kernel,
    out_type=jax.ShapeDtypeStruct((_NW * _NH,), jnp.float32),
    mesh=_mesh,
    compiler_params=pltpu.CompilerParams(needs_layout_passes=False),
    scratch_types=[
        pltpu.VMEM((_NAOS,), jnp.float32),
        pltpu.VMEM((3 * _SOA + 16,), jnp.float32),
        pltpu.VMEM((16 * _REG,), jnp.float32),
        pltpu.VMEM((16,), jnp.int32),
        pltpu.VMEM((16 * _NH,), jnp.float32),
        pltpu.VMEM((_NH,), jnp.float32),
    ],
)
def _sc_hist(coords_hbm, out_hbm, cvm, soa, buf, cntv, hist, outv):
    wid = lax.axis_index("s") * 2 + lax.axis_index("c")
    pltpu.sync_copy(coords_hbm, cvm)
    iota = lax.iota(jnp.int32, 16)
    iota3 = iota * 3
    zero16 = jnp.zeros((16,), jnp.float32)

    def zh(k, carry):
        hist[pl.ds(k * 16, 16)] = zero16
        return carry

    lax.fori_loop(0, 16 * _NH // 16, zh, 0)

    def tr(k, carry):
        p = k // 64
        b = (k // 32) % 2
        c = k % 32
        gi = iota3 + (b * 3 * _NA + c * 48 + p)
        gi = jnp.minimum(gi, _NAOS - 1)
        soa[pl.ds(p * _SOA + b * _NAP + c * 16, 16)] = (
            plsc.load_gather(cvm, [gi]))
        return carry

    lax.fori_loop(0, 192, tr, 0)

    def wrap_sq(d):
        a = jnp.abs(d)
        b = 1.0 - a
        return jnp.minimum(a * a, b * b)

    lane_base = iota * _REG

    def one_batch(b, cnt0):
        base = b * _NAP
        nrows = (_NA - 1 - wid) // _NW + 1
        nrows2 = (nrows + 1) // 2

        def row_body(ri, cnt):
            ia = wid + 2 * _NW * ri
            ib = ia + _NW
            has_b = ib <= _NA - 1
            civ = jnp.full((16,), base + ia, jnp.int32)
            xa = plsc.load_gather(soa, [civ])
            ya = plsc.load_gather(soa, [civ + _SOA])
            za = plsc.load_gather(soa, [civ + 2 * _SOA])
            cjv = jnp.full((16,), base + jnp.where(has_b, ib, ia), jnp.int32)
            xb = plsc.load_gather(soa, [cjv])
            yb = plsc.load_gather(soa, [cjv + _SOA])
            zb = plsc.load_gather(soa, [cjv + 2 * _SOA])
            ihi = jnp.where(has_b, ib, ia)
            nj2 = (ihi + 31) // 32

            def jv_body(jv2, cnt1):
                off = base + jv2 * 32
                ja = jv2 * 32 + iota
                cx0 = soa[pl.ds(off, 16)]
                cx1 = soa[pl.ds(off + 16, 16)]
                cy0 = soa[pl.ds(off + _SOA, 16)]
                cy1 = soa[pl.ds(off + _SOA + 16, 16)]
                cz0 = soa[pl.ds(off + 2 * _SOA, 16)]
                cz1 = soa[pl.ds(off + 2 * _SOA + 16, 16)]
                d_a0 = wrap_sq(xa - cx0) + wrap_sq(ya - cy0) + wrap_sq(za - cz0)
                d_a1 = wrap_sq(xa - cx1) + wrap_sq(ya - cy1) + wrap_sq(za - cz1)
                d_b0 = wrap_sq(xb - cx0) + wrap_sq(yb - cy0) + wrap_sq(zb - cz0)
                d_b1 = wrap_sq(xb - cx1) + wrap_sq(yb - cy1) + wrap_sq(zb - cz1)
                m_a0 = (d_a0 < _R2T) & (d_a0 != 0.0) & (ja < ia)
                m_a1 = (d_a1 < _R2T) & (d_a1 != 0.0) & (ja + 16 < ia)
                m_b0 = (d_b0 < _R2T) & (d_b0 != 0.0) & (ja < ib) & has_b
                m_b1 = (d_b1 < _R2T) & (d_b1 != 0.0) & (ja + 16 < ib) & has_b
                plsc.store_scatter(buf, [lane_base + cnt1], d_a0, mask=m_a0)
                c2 = cnt1 + m_a0.astype(jnp.int32)
                plsc.store_scatter(buf, [lane_base + c2], d_a1, mask=m_a1)
                c3 = c2 + m_a1.astype(jnp.int32)
                plsc.store_scatter(buf, [lane_base + c3], d_b0, mask=m_b0)
                c4 = c3 + m_b0.astype(jnp.int32)
                plsc.store_scatter(buf, [lane_base + c4], d_b1, mask=m_b1)
                return c4 + m_b1.astype(jnp.int32)

            return lax.fori_loop(0, nj2, jv_body, cnt)

        return lax.fori_loop(0, nrows2, row_body, cnt0)

    cnt = one_batch(0, jnp.zeros((16,), jnp.int32))
    cnt = one_batch(1, cnt)
    cntv[pl.ds(0, 16)] = cnt


    def smear(dsq, valid):
        bits = plsc.bitcast(dsq, jnp.int32)
        y = plsc.bitcast(
            jnp.int32(0x5F3759DF) - lax.shift_right_logical(bits, 1),
            jnp.float32)
        for _ in range(3):
            y = y * (1.5 - 0.5 * dsq * y * y)
        t = dsq * y * _INVW
        i0 = (t + 0.5).astype(jnp.int32)
        i0 = jnp.minimum(jnp.maximum(i0, 0), _NBINS + _J)
        f = t - i0.astype(jnp.float32)
        base_idx = iota * _NH + i0
        for jj in range(2 * _J + 1):
            a = f + float(_J - jj)
            wv = jnp.exp(-0.5 * a * a)
            plsc.addupdate_scatter(hist, [base_idx + jj], wv, mask=valid)

    def lane_loop(l, carry):
        clv = plsc.load_gather(cntv, [jnp.full((16,), l, jnp.int32)])
        nvl2 = (clv[0] + 31) // 32
        lb = l * _REG

        def pv(v, carry2):
            off = lb + v * 32
            dsq_a = buf[pl.ds(off, 16)]
            dsq_b = buf[pl.ds(off + 16, 16)]
            smear(dsq_a, (v * 32 + iota) < clv)
            smear(dsq_b, (v * 32 + 16 + iota) < clv)
            return carry2

        return lax.fori_loop(0, nvl2, pv, carry)

    lax.fori_loop(0, 16, lane_loop, jnp.int32(0))

    def red(c, carry):
        acc = hist[pl.ds(c * 16, 16)]
        for r in range(1, 16):
            acc = acc + hist[pl.ds(r * _NH + c * 16, 16)]
        outv[pl.ds(c * 16, 16)] = acc
        return carry

    lax.fori_loop(0, 8, red, 0)
    pltpu.sync_copy(outv, out_hbm.at[pl.ds(wid * _NH, _NH)])


def _finish_body(part_ref, volr_ref, count_ref, rdf_ref):
    x = part_ref[...]
    acc = x[0:_NH]
    for r in range(1, _NW):
        acc = acc + x[r * _NH:(r + 1) * _NH]
    w = acc[_J:_J + _NBINS]
    count = w / jnp.sum(w)
    count_ref[...] = count
    rdf_ref[...] = count * volr_ref[...]


def kernel(xyz):
    coords = xyz.reshape(-1)
    part = _sc_hist(coords)
    bins = jnp.linspace(0.0, _CUTOFF, _NBINS + 1)
    npbins = np.linspace(0.0, _CUTOFF, _NBINS + 1, dtype=np.float32)
    vol_bins = 4.0 * np.pi / 3.0 * (npbins[1:] ** 3 - npbins[:-1] ** 3)
    V = 4.0 / 3.0 * np.pi * _CUTOFF ** 3
    volr = jnp.asarray(V / vol_bins, dtype=jnp.float32)
    count, rdf_out = pl.pallas_call(
        _finish_body,
        out_shape=(jax.ShapeDtypeStruct((_NBINS,), jnp.float32),
                   jax.ShapeDtypeStruct((_NBINS,), jnp.float32)),
    )(part, volr)
    return (count, bins, rdf_out)
